# baseline (speedup 1.0000x reference)
import functools

import jax
import jax.numpy as jnp
from jax import lax
from jax.experimental import pallas as pl
from jax.experimental.pallas import tpu as pltpu
from jax.experimental.pallas import tpu_sc as plsc

NUM_ELEMENTS = 120
EMBED_DIM = 64
N_ATOMS = 100000

NC = 2
NS = 16
NW = NC * NS

CHUNK = 128
PER_W = 3200
PER_LAST = N_ATOMS - (NW - 1) * PER_W
NCH = PER_W // CHUNK
NCH_L = PER_LAST // CHUNK
TAIL_L = PER_LAST - NCH_L * CHUNK
NBUF = 3
TPITCH = CHUNK + 1


def _gather_body(table_hbm, idx_hbm, outT_hbm, idx_v, table_sh, r_v, t_v,
                 gsem, osem, tsem):
    sid = lax.axis_index("s")
    wid = sid * NC + lax.axis_index("c")
    base = wid * PER_W
    @pl.when(sid == 0)
    def _():
        pltpu.sync_copy(table_hbm, table_sh)

    iota16 = lax.iota(jnp.int32, 16)
    jidx = [iota16 + 16 * jb for jb in range(4)]

    def transpose_col_block(rbuf, tbuf, i):
        ivec = iota16 * 0 + i
        for jb in range(4):
            v = rbuf[i, pl.ds(16 * jb, 16)]
            plsc.store_scatter(tbuf, [jidx[jb], ivec], v)

    def pipeline(n_idx, nch, tailw):
        pltpu.sync_copy(idx_hbm.at[pl.ds(base, n_idx)],
                        idx_v.at[pl.ds(0, n_idx)])
        plsc.subcore_barrier()

        def mk_gather(c):
            return pltpu.make_async_copy(
                table_sh.at[idx_v.at[pl.ds(c * CHUNK, CHUNK)]],
                r_v.at[c % NBUF],
                gsem.at[c % NBUF],
            )

        def mk_out(c):
            return pltpu.make_async_copy(
                t_v.at[c % 2, :, pl.ds(0, CHUNK)],
                outT_hbm.at[:, pl.ds(base + c * CHUNK, CHUNK)],
                osem.at[c % 2],
            )

        for b in range(NBUF - 1):
            mk_gather(b).start()

        def step(c, carry):
            mk_gather(c).wait()

            @pl.when(c + NBUF - 1 < nch)
            def _():
                mk_gather(c + NBUF - 1).start()

            @pl.when(c >= 2)
            def _():
                mk_out(c - 2).wait()

            rbuf = r_v.at[c % NBUF]
            tbuf = t_v.at[c % 2]

            @plsc.parallel_loop(0, CHUNK, unroll=16)
            def _(i):
                transpose_col_block(rbuf, tbuf, i)

            mk_out(c).start()
            return carry

        lax.fori_loop(0, nch, step, 0, unroll=False)

        mk_out(nch - 1).wait()
        @pl.when(nch >= 2)
        def _():
            mk_out(nch - 2).wait()

        if tailw:
            pltpu.make_async_copy(
                table_sh.at[idx_v.at[pl.ds(nch * CHUNK, tailw)]],
                r_v.at[0, pl.ds(0, tailw)],
                tsem,
            ).start()
            pltpu.make_async_copy(
                table_sh.at[idx_v.at[pl.ds(nch * CHUNK, tailw)]],
                r_v.at[0, pl.ds(0, tailw)],
                tsem,
            ).wait()

            @plsc.parallel_loop(0, tailw, unroll=8)
            def _(i):
                transpose_col_block(r_v.at[0], t_v.at[0], i)
            pltpu.sync_copy(
                t_v.at[0, :, pl.ds(0, CHUNK)],
                outT_hbm.at[:, pl.ds(base + nch * CHUNK, CHUNK)],
            )

    @pl.when(wid < NW - 1)
    def _():
        pipeline(PER_W, NCH, 0)

    @pl.when(wid == NW - 1)
    def _():
        pipeline(PER_LAST, NCH_L, TAIL_L)


@jax.jit
def _sc_gather(table, idx):
    mesh = plsc.VectorSubcoreMesh(core_axis_name="c", subcore_axis_name="s")
    f = functools.partial(
        pl.kernel,
        out_type=jax.ShapeDtypeStruct((EMBED_DIM, N_ATOMS), jnp.float32),
        mesh=mesh,
        scratch_types=[
            pltpu.VMEM((PER_W,), jnp.int32),
            pltpu.VMEM_SHARED((NUM_ELEMENTS, 128), jnp.float32),
            pltpu.VMEM((NBUF, CHUNK, 128), jnp.float32),
            pltpu.VMEM((2, EMBED_DIM, TPITCH), jnp.float32),
            pltpu.SemaphoreType.DMA((NBUF,)),
            pltpu.SemaphoreType.DMA((2,)),
            pltpu.SemaphoreType.DMA,
        ],
        compiler_params=pltpu.CompilerParams(use_tc_tiling_on_sc=True,
                                             needs_layout_passes=False),
    )(_gather_body)
    return f(table, idx)


def kernel(atomic_numbers, table):
    table128 = jnp.pad(table, ((0, 0), (0, 128 - EMBED_DIM)))
    return _sc_gather(table128, atomic_numbers.astype(jnp.int32)).T


# speedup vs baseline: 1.1671x; 1.1671x over previous
import functools

import jax
import jax.numpy as jnp
from jax import lax
from jax.experimental import pallas as pl
from jax.experimental.pallas import tpu as pltpu
from jax.experimental.pallas import tpu_sc as plsc

NUM_ELEMENTS = 120
EMBED_DIM = 64
N_ATOMS = 100000

NC = 2
NS = 16
NW = NC * NS

PER_W = ((N_ATOMS // NW + 7) // 8) * 8
PER_LAST = N_ATOMS - (NW - 1) * PER_W

CHUNK = 256
NCH = PER_W // CHUNK
TAIL = PER_W - NCH * CHUNK
NCH_L = PER_LAST // CHUNK
TAIL_L = PER_LAST - NCH_L * CHUNK
NBUF = 3


def _gather_body(table_hbm, idx_hbm, out_hbm, idx_v, table_sh, rows_v, tail_v,
                 gsem, ssem, tsem):
    sid = lax.axis_index("s")
    wid = sid * NC + lax.axis_index("c")
    base = wid * PER_W
    @pl.when(sid == 0)
    def _():
        pltpu.sync_copy(table_hbm, table_sh)

    def pipeline(n_idx, nch, tail):
        pltpu.sync_copy(idx_hbm.at[pl.ds(base, n_idx)],
                        idx_v.at[pl.ds(0, n_idx)])
        plsc.subcore_barrier()

        def mk_gather(c):
            return pltpu.make_async_copy(
                table_sh.at[idx_v.at[pl.ds(c * CHUNK, CHUNK)]],
                rows_v.at[c % NBUF],
                gsem.at[c % NBUF],
            )

        def mk_store(c):
            return pltpu.make_async_copy(
                rows_v.at[c % NBUF],
                out_hbm.at[pl.ds(base + c * CHUNK, CHUNK)],
                ssem.at[c % NBUF],
            )

        def mk_tail_gather():
            return pltpu.make_async_copy(
                table_sh.at[idx_v.at[pl.ds(nch * CHUNK, tail)]],
                tail_v.at[pl.ds(0, tail)],
                tsem,
            )

        mk_tail_gather().start()
        for b in range(NBUF - 1):
            mk_gather(b).start()

        def step(j, carry):
            mk_gather(j).wait()
            mk_store(j).start()

            @pl.when(j >= 1)
            def _():
                mk_store(j - 1).wait()

            @pl.when(j + NBUF - 1 < nch)
            def _():
                mk_gather(j + NBUF - 1).start()

            return carry

        lax.fori_loop(0, nch, step, 0, unroll=False)

        mk_store(nch - 1).wait()
        mk_tail_gather().wait()
        pltpu.sync_copy(tail_v.at[pl.ds(0, tail)],
                        out_hbm.at[pl.ds(base + nch * CHUNK, tail)])

    @pl.when(wid < NW - 1)
    def _():
        pipeline(PER_W, NCH, TAIL)

    @pl.when(wid == NW - 1)
    def _():
        pipeline(PER_LAST, NCH_L, TAIL_L)


@jax.jit
def _sc_gather(table, idx):
    mesh = plsc.VectorSubcoreMesh(core_axis_name="c", subcore_axis_name="s")
    f = functools.partial(
        pl.kernel,
        out_type=jax.ShapeDtypeStruct((N_ATOMS, EMBED_DIM), jnp.float32),
        mesh=mesh,
        scratch_types=[
            pltpu.VMEM((PER_W,), jnp.int32),
            pltpu.VMEM_SHARED((NUM_ELEMENTS, EMBED_DIM), jnp.float32),
            pltpu.VMEM((NBUF, CHUNK, EMBED_DIM), jnp.float32),
            pltpu.VMEM((TAIL_L, EMBED_DIM), jnp.float32),
            pltpu.SemaphoreType.DMA((NBUF,)),
            pltpu.SemaphoreType.DMA((NBUF,)),
            pltpu.SemaphoreType.DMA,
        ],
        compiler_params=pltpu.CompilerParams(use_tc_tiling_on_sc=False),
    )(_gather_body)
    return f(table, idx)


def kernel(atomic_numbers, table):
    return _sc_gather(table, atomic_numbers.astype(jnp.int32))
